# baseline (speedup 1.0000x reference)
import jax
import jax.numpy as jnp
from jax import lax
from jax.experimental import pallas as pl
from jax.experimental.pallas import tpu as pltpu
from jax.experimental.pallas import tpu_sc as plsc

OUT_N = 100000
OUT_PAD = 100352
NUM_E = 6400000
NC, NS, L = 2, 16, 16
NW = NC * NS
CCOL = 3200
NCH = NUM_E // CCOL
NBUF = 4
KPW = 64
VPC = CCOL // L
UNROLL = 10
ZUNROLL = 8
NZERO = OUT_PAD // L

P2_W = 3200
P2_WLAST = OUT_PAD - (NW - 1) * P2_W
P2_BUF = 1664


def _sc_hist_body(edge_hbm, partial_hbm, buf0, buf1, buf2, buf3, counts_v,
                  sem0, sem1, sem2, sem3):
    wid = lax.axis_index("s") * NC + lax.axis_index("c")
    bufs = (buf0, buf1, buf2, buf3)
    sems = (sem0, sem1, sem2, sem3)

    zeros16 = jnp.zeros((L,), jnp.float32)
    ones16 = jnp.ones((L,), jnp.float32)

    def _col(k):
        q = jnp.minimum(k * NW + wid, NCH - 1)
        return pl.multiple_of(q * CCOL, CCOL)

    def _start(k, buf, sem):
        pltpu.async_copy(edge_hbm.at[:, pl.ds(_col(k), CCOL)], buf, sem)

    for b in range(NBUF):
        _start(b, bufs[b], sems[b])

    @plsc.parallel_loop(0, NZERO, 1, unroll=ZUNROLL)
    def _clear(j):
        counts_v[pl.ds(j * L, L)] = zeros16

    def _wait(buf, sem):
        pltpu.make_async_copy(edge_hbm.at[:, pl.ds(0, CCOL)], buf, sem).wait()

    def _scatter(k, buf):
        live = (k * NW + wid) < NCH
        mask = jnp.full((L,), live)

        @plsc.parallel_loop(0, VPC, 1, unroll=UNROLL)
        def _vec(i):
            idx = buf[1, pl.ds(i * L, L)]
            plsc.addupdate_scatter(counts_v, [idx], ones16, mask=mask)

    def _outer(j, c):
        k0 = NBUF * j
        for b in range(NBUF):
            k = k0 + b
            _wait(bufs[b], sems[b])
            _scatter(k, bufs[b])

            @pl.when(k + NBUF < KPW)
            def _():
                _start(k + NBUF, bufs[b], sems[b])

        return c

    lax.fori_loop(0, KPW // NBUF, _outer, 0)

    pltpu.sync_copy(counts_v, partial_hbm.at[wid])


def _sc_reduce_body(partial_hbm, out_hbm, bufa_v, bufb_v, acc_v, sema, semb):
    wid = lax.axis_index("s") * NC + lax.axis_index("c")
    start = wid * P2_W

    def _half_start(S, W, buf, sem):
        pltpu.async_copy(partial_hbm.at[:, pl.ds(S, W)],
                         buf.at[:, pl.ds(0, W)], sem)

    def _half_acc(S, W, buf, sem):
        pltpu.make_async_copy(partial_hbm.at[:, pl.ds(S, W)],
                              buf.at[:, pl.ds(0, W)], sem).wait()

        @plsc.parallel_loop(0, W // L, 1, unroll=2)
        def _g(i):
            col = i * L
            a = buf[0, pl.ds(col, L)]
            for r in range(1, NW):
                a = a + buf[r, pl.ds(col, L)]
            acc_v[pl.ds(col, L)] = a

        pltpu.sync_copy(acc_v.at[pl.ds(0, W)], out_hbm.at[pl.ds(S, W)])

    def _reduce(W):
        wa = (W // 2 // 128) * 128
        wb = W - wa
        _half_start(start, wa, bufa_v, sema)
        _half_start(start + wa, wb, bufb_v, semb)
        _half_acc(start, wa, bufa_v, sema)
        _half_acc(start + wa, wb, bufb_v, semb)

    @pl.when(wid < NW - 1)
    def _():
        _reduce(P2_W)

    @pl.when(wid == NW - 1)
    def _():
        _reduce(P2_WLAST)


def kernel(edge_index, num_nodes, num_edges):
    del num_nodes, num_edges

    mesh = plsc.VectorSubcoreMesh(core_axis_name="c", subcore_axis_name="s")
    params = pltpu.CompilerParams(needs_layout_passes=False)

    partials = pl.kernel(
        _sc_hist_body,
        out_type=jax.ShapeDtypeStruct((NW, OUT_PAD), jnp.float32),
        mesh=mesh,
        compiler_params=params,
        scratch_types=[
            pltpu.VMEM((2, CCOL), jnp.int32),
            pltpu.VMEM((2, CCOL), jnp.int32),
            pltpu.VMEM((2, CCOL), jnp.int32),
            pltpu.VMEM((2, CCOL), jnp.int32),
            pltpu.VMEM((OUT_PAD,), jnp.float32),
            pltpu.SemaphoreType.DMA,
            pltpu.SemaphoreType.DMA,
            pltpu.SemaphoreType.DMA,
            pltpu.SemaphoreType.DMA,
        ],
    )(edge_index)

    counts = pl.kernel(
        _sc_reduce_body,
        out_type=jax.ShapeDtypeStruct((OUT_PAD,), jnp.float32),
        mesh=mesh,
        compiler_params=params,
        scratch_types=[
            pltpu.VMEM((NW, P2_BUF), jnp.float32),
            pltpu.VMEM((NW, P2_BUF), jnp.float32),
            pltpu.VMEM((P2_BUF,), jnp.float32),
            pltpu.SemaphoreType.DMA,
            pltpu.SemaphoreType.DMA,
        ],
    )(partials)
    return counts[:OUT_N]


# speedup vs baseline: 1.0220x; 1.0220x over previous
import jax
import jax.numpy as jnp
from jax import lax
from jax.experimental import pallas as pl
from jax.experimental.pallas import tpu as pltpu
from jax.experimental.pallas import tpu_sc as plsc

OUT_N = 100000
OUT_PAD = 100352
NUM_E = 6400000
NC, NS, L = 2, 16, 16
NW = NC * NS
CCOL = 3200
NCH = NUM_E // CCOL
NBUF = 4
KPW = 64
VPC = CCOL // L
UNROLL = 10
ZUNROLL = 8
NZERO = OUT_PAD // L

P2_W = 3200
P2_WLAST = OUT_PAD - (NW - 1) * P2_W


def _sc_hist_body(edge_hbm, partial_hbm, buf0, buf1, buf2, buf3, counts_v,
                  sem0, sem1, sem2, sem3):
    wid = lax.axis_index("s") * NC + lax.axis_index("c")
    bufs = (buf0, buf1, buf2, buf3)
    sems = (sem0, sem1, sem2, sem3)

    zeros16 = jnp.zeros((L,), jnp.float32)
    ones16 = jnp.ones((L,), jnp.float32)

    def _col(k):
        q = jnp.minimum(k * NW + wid, NCH - 1)
        return pl.multiple_of(q * CCOL, CCOL)

    def _start(k, buf, sem):
        pltpu.async_copy(edge_hbm.at[:, pl.ds(_col(k), CCOL)], buf, sem)

    for b in range(NBUF):
        _start(b, bufs[b], sems[b])

    @plsc.parallel_loop(0, NZERO, 1, unroll=ZUNROLL)
    def _clear(j):
        counts_v[pl.ds(j * L, L)] = zeros16

    def _wait(buf, sem):
        pltpu.make_async_copy(edge_hbm.at[:, pl.ds(0, CCOL)], buf, sem).wait()

    def _scatter(k, buf):
        live = (k * NW + wid) < NCH
        mask = jnp.full((L,), live)

        @plsc.parallel_loop(0, VPC, 1, unroll=UNROLL)
        def _vec(i):
            idx = buf[1, pl.ds(i * L, L)]
            plsc.addupdate_scatter(counts_v, [idx], ones16, mask=mask)

    def _outer(j, c):
        k0 = NBUF * j
        for b in range(NBUF):
            k = k0 + b
            _wait(bufs[b], sems[b])
            _scatter(k, bufs[b])

            @pl.when(k + NBUF < KPW)
            def _():
                _start(k + NBUF, bufs[b], sems[b])

        return c

    lax.fori_loop(0, KPW // NBUF, _outer, 0)

    pltpu.sync_copy(counts_v, partial_hbm.at[wid])


def _sc_reduce_body(partial_hbm, out_hbm, buf_v, acc_v):
    wid = lax.axis_index("s") * NC + lax.axis_index("c")
    start = wid * P2_W

    def _reduce(W):
        pltpu.sync_copy(partial_hbm.at[:, pl.ds(start, W)],
                        buf_v.at[:, pl.ds(0, W)])

        @plsc.parallel_loop(0, W // L, 1, unroll=2)
        def _g(i):
            col = i * L
            a = buf_v[0, pl.ds(col, L)]
            for r in range(1, NW):
                a = a + buf_v[r, pl.ds(col, L)]
            acc_v[pl.ds(col, L)] = a

        pltpu.sync_copy(acc_v.at[pl.ds(0, W)], out_hbm.at[pl.ds(start, W)])

    @pl.when(wid < NW - 1)
    def _():
        _reduce(P2_W)

    @pl.when(wid == NW - 1)
    def _():
        _reduce(P2_WLAST)


def kernel(edge_index, num_nodes, num_edges):
    del num_nodes, num_edges

    mesh = plsc.VectorSubcoreMesh(core_axis_name="c", subcore_axis_name="s")
    params = pltpu.CompilerParams(needs_layout_passes=False)

    partials = pl.kernel(
        _sc_hist_body,
        out_type=jax.ShapeDtypeStruct((NW, OUT_PAD), jnp.float32),
        mesh=mesh,
        compiler_params=params,
        scratch_types=[
            pltpu.VMEM((2, CCOL), jnp.int32),
            pltpu.VMEM((2, CCOL), jnp.int32),
            pltpu.VMEM((2, CCOL), jnp.int32),
            pltpu.VMEM((2, CCOL), jnp.int32),
            pltpu.VMEM((OUT_PAD,), jnp.float32),
            pltpu.SemaphoreType.DMA,
            pltpu.SemaphoreType.DMA,
            pltpu.SemaphoreType.DMA,
            pltpu.SemaphoreType.DMA,
        ],
    )(edge_index)

    counts = pl.kernel(
        _sc_reduce_body,
        out_type=jax.ShapeDtypeStruct((OUT_PAD,), jnp.float32),
        mesh=mesh,
        compiler_params=params,
        scratch_types=[
            pltpu.VMEM((NW, P2_W), jnp.float32),
            pltpu.VMEM((P2_W,), jnp.float32),
        ],
    )(partials)
    return counts[:OUT_N]
